# baseline (speedup 1.0000x reference)
import numpy as np

import jax
import jax.numpy as jnp
from jax import lax
from jax.experimental import pallas as pl
from jax.experimental.pallas import tpu as pltpu
from jax.experimental.pallas import tpu_sc as plsc

_L = 200
_D = 128
_B = 1024
_N = _B * _L
_NC, _NS = 2, 16
_NW = _NC * _NS
_SENT_W = _B // _NW
_SPLIT = 128
_NBUF = 3


def _pos_encoding_np():
    pos = np.arange(_L, dtype=np.float64)[:, None]
    i = np.arange(0, _D, 2, dtype=np.float64)
    div = np.exp(-np.log(10000.0) * i / _D)
    pe = np.zeros((_L, _D), dtype=np.float32)
    pe[:, 0::2] = np.sin(pos * div).astype(np.float32)
    pe[:, 1::2] = np.cos(pos * div).astype(np.float32)
    return pe


def _embed_body(table_hbm, x_hbm, pe_hbm, out_hbm,
                idx_v, pe_v, buf0, buf1, buf2,
                gl0, gl1, gl2, gh0, gh1, gh2, osem0, osem1, osem2):
    wid = lax.axis_index("s") * _NC + lax.axis_index("c")
    b0 = wid * _SENT_W

    pltpu.sync_copy(x_hbm.at[pl.ds(b0, _SENT_W)], idx_v)
    pltpu.sync_copy(pe_hbm, pe_v)

    bufs = (buf0, buf1, buf2)
    glsems = (gl0, gl1, gl2)
    ghsems = (gh0, gh1, gh2)
    osems = (osem0, osem1, osem2)

    def g_lo(j, s):
        return (table_hbm.at[idx_v.at[j, pl.ds(0, _SPLIT)]],
                bufs[s].at[pl.ds(0, _SPLIT)], glsems[s])

    def g_hi(j, s):
        return (table_hbm.at[idx_v.at[j, pl.ds(_SPLIT, _L - _SPLIT)]],
                bufs[s].at[pl.ds(_SPLIT, _L - _SPLIT)], ghsems[s])

    def g_start(j, s):
        pltpu.async_copy(*g_lo(j, s))
        pltpu.async_copy(*g_hi(j, s))

    def o_start(j, s):
        pltpu.async_copy(bufs[s], out_hbm.at[b0 + j], osems[s])

    def o_wait(j, s):
        pltpu.make_async_copy(bufs[s], out_hbm.at[b0 + j], osems[s]).wait()

    def add_rows(s, r0, nr):
        buf = bufs[s]

        @pl.loop(r0, r0 + nr, unroll=2)
        def _row(r):
            for c in range(_D // 16):
                sl = pl.ds(c * 16, 16)
                plsc.addupdate(buf.at[r, sl], pe_v[r, sl])

    def body(j, s, *, wait_out=True, start_next=True):
        if wait_out:
            o_wait(j - 2, (s - 2) % _NBUF)
        if start_next:
            g_start(j + 1, (s + 1) % _NBUF)
        pltpu.make_async_copy(*g_lo(j, s)).wait()
        add_rows(s, 0, _SPLIT)
        pltpu.make_async_copy(*g_hi(j, s)).wait()
        add_rows(s, _SPLIT, _L - _SPLIT)
        o_start(j, s)

    g_start(0, 0)
    body(0, 0, wait_out=False)
    body(1, 1, wait_out=False)

    @pl.loop(2, _SENT_W - 3, step=_NBUF)
    def _trip(jj):
        for off in range(_NBUF):
            body(jj + off, (2 + off) % _NBUF)

    body(_SENT_W - 3, (_SENT_W - 3) % _NBUF)
    body(_SENT_W - 2, (_SENT_W - 2) % _NBUF)
    body(_SENT_W - 1, (_SENT_W - 1) % _NBUF, start_next=False)
    o_wait(_SENT_W - 2, (_SENT_W - 2) % _NBUF)
    o_wait(_SENT_W - 1, (_SENT_W - 1) % _NBUF)


def kernel(x, table):
    pe = jnp.asarray(_pos_encoding_np())
    mesh = plsc.VectorSubcoreMesh(core_axis_name="c", subcore_axis_name="s")
    run = pl.kernel(
        _embed_body,
        out_type=jax.ShapeDtypeStruct((_B, _L, _D), jnp.float32),
        mesh=mesh,
        scratch_types=[
            pltpu.VMEM((_SENT_W, _L), jnp.int32),
            pltpu.VMEM((_L, _D), jnp.float32),
            pltpu.VMEM((_L, _D), jnp.float32),
            pltpu.VMEM((_L, _D), jnp.float32),
            pltpu.VMEM((_L, _D), jnp.float32),
        ] + [pltpu.SemaphoreType.DMA] * 9,
    )
    return run(table, x.astype(jnp.int32), pe)


# speedup vs baseline: 1.0208x; 1.0208x over previous
import numpy as np

import jax
import jax.numpy as jnp
from jax import lax
from jax.experimental import pallas as pl
from jax.experimental.pallas import tpu as pltpu
from jax.experimental.pallas import tpu_sc as plsc

_L = 200
_D = 128
_B = 1024
_NC, _NS = 2, 16
_NW = _NC * _NS
_SENT_W = _B // _NW
_LO = 128
_HI = _L - _LO
_NH = 2 * _SENT_W


def _pos_encoding_np():
    pos = np.arange(_L, dtype=np.float64)[:, None]
    i = np.arange(0, _D, 2, dtype=np.float64)
    div = np.exp(-np.log(10000.0) * i / _D)
    pe = np.zeros((_L, _D), dtype=np.float32)
    pe[:, 0::2] = np.sin(pos * div).astype(np.float32)
    pe[:, 1::2] = np.cos(pos * div).astype(np.float32)
    return pe


def _embed_body(table_hbm, x_hbm, pe_hbm, out_hbm,
                idx_v, pe_v, lo0, lo1, lo2, hi0, hi1, hi2, *sems):
    wid = lax.axis_index("s") * _NC + lax.axis_index("c")
    b0 = wid * _SENT_W

    pltpu.sync_copy(x_hbm.at[pl.ds(b0, _SENT_W)], idx_v)
    pltpu.sync_copy(pe_hbm, pe_v)

    lobufs, hibufs = (lo0, lo1, lo2), (hi0, hi1, hi2)
    glsems, ghsems = sems[0:3], sems[3:6]
    olsems, ohsems = sems[6:9], sems[9:12]

    def parts(h, s, par):
        j = h // 2
        r0 = (b0 + j) * _L
        if par == 0:
            return (table_hbm.at[idx_v.at[j, pl.ds(0, _LO)]],
                    lobufs[s], out_hbm.at[pl.ds(r0, _LO)],
                    glsems[s], olsems[s])
        return (table_hbm.at[idx_v.at[j, pl.ds(_LO, _HI)]],
                hibufs[s], out_hbm.at[pl.ds(r0 + _LO, _HI)],
                ghsems[s], ohsems[s])

    def g_start(h, s, par):
        src, buf, _, gsem, _ = parts(h, s, par)
        pltpu.async_copy(src, buf, gsem)

    def g_wait(h, s, par):
        src, buf, _, gsem, _ = parts(h, s, par)
        pltpu.make_async_copy(src, buf, gsem).wait()

    def o_start(h, s, par):
        _, buf, dst, _, osem = parts(h, s, par)
        pltpu.async_copy(buf, dst, osem)

    def o_wait(h, s, par):
        _, buf, dst, _, osem = parts(h, s, par)
        pltpu.make_async_copy(buf, dst, osem).wait()

    def add_pe(s, par):
        buf = (lobufs if par == 0 else hibufs)[s]
        r0, nr = (0, _LO) if par == 0 else (_LO, _HI)

        @pl.loop(0, nr)
        def _row(r):
            for c in range(_D // 16):
                sl = pl.ds(c * 16, 16)
                plsc.addupdate(buf.at[r, sl], pe_v[r0 + r, sl])

    def body(h, s, par, *, wait_out=True, start_next=True):
        if wait_out:
            o_wait(h - 4, (s + 1) % 3, par)
        if start_next:
            g_start(h + 2, (s + 1) % 3, par)
        g_wait(h, s, par)
        add_pe(s, par)
        o_start(h, s, par)

    def sp(hh):
        return (hh // 2) % 3, hh % 2

    g_start(0, 0, 0)
    g_start(1, 0, 1)
    for hh in range(4):
        body(hh, *sp(hh), wait_out=False)

    @pl.loop(4, _NH - 6, step=6)
    def _six(h):
        for off in range(6):
            body(h + off, *sp(4 + off))

    for hh in range(_NH - 6, _NH):
        body(hh, *sp(hh), start_next=(hh + 2 < _NH))
    for hh in range(_NH - 4, _NH):
        o_wait(hh, *sp(hh))


def kernel(x, table):
    pe = jnp.asarray(_pos_encoding_np())
    mesh = plsc.VectorSubcoreMesh(core_axis_name="c", subcore_axis_name="s")
    run = pl.kernel(
        _embed_body,
        out_type=jax.ShapeDtypeStruct((_B * _L, _D), jnp.float32),
        mesh=mesh,
        scratch_types=[
            pltpu.VMEM((_SENT_W, _L), jnp.int32),
            pltpu.VMEM((_L, _D), jnp.float32),
            pltpu.VMEM((_LO, _D), jnp.float32),
            pltpu.VMEM((_LO, _D), jnp.float32),
            pltpu.VMEM((_LO, _D), jnp.float32),
            pltpu.VMEM((_HI, _D), jnp.float32),
            pltpu.VMEM((_HI, _D), jnp.float32),
            pltpu.VMEM((_HI, _D), jnp.float32),
        ] + [pltpu.SemaphoreType.DMA] * 12,
    )
    return run(table, x.astype(jnp.int32), pe).reshape(_B, _L, _D)


# speedup vs baseline: 1.0305x; 1.0095x over previous
import numpy as np

import jax
import jax.numpy as jnp
from jax import lax
from jax.experimental import pallas as pl
from jax.experimental.pallas import tpu as pltpu
from jax.experimental.pallas import tpu_sc as plsc

_L = 200
_D = 128
_B = 1024
_NC, _NS = 2, 16
_NW = _NC * _NS
_SENT_W = _B // _NW
_LO = 128
_HI = _L - _LO
_NH = 2 * _SENT_W


def _pos_encoding_np():
    pos = np.arange(_L, dtype=np.float64)[:, None]
    i = np.arange(0, _D, 2, dtype=np.float64)
    div = np.exp(-np.log(10000.0) * i / _D)
    pe = np.zeros((_L, _D), dtype=np.float32)
    pe[:, 0::2] = np.sin(pos * div).astype(np.float32)
    pe[:, 1::2] = np.cos(pos * div).astype(np.float32)
    return pe


def _embed_body(table_hbm, x_hbm, pe_hbm, out_hbm,
                idx_v, pe_v, lo0, lo1, lo2, hi0, hi1, hi2, *sems):
    wid = lax.axis_index("s") * _NC + lax.axis_index("c")
    b0 = wid * _SENT_W

    lobufs, hibufs = (lo0, lo1, lo2), (hi0, hi1, hi2)
    glsems, ghsems = sems[0:3], sems[3:6]
    olsems, ohsems = sems[6:9], sems[9:12]
    pesem = sems[12]

    pltpu.sync_copy(x_hbm.at[pl.ds(b0, _SENT_W)], idx_v)
    pe_copy = pltpu.async_copy(pe_hbm, pe_v, pesem)

    def parts(h, s, par):
        j = h // 2
        r0 = (b0 + j) * _L
        if par == 0:
            return (table_hbm.at[idx_v.at[j, pl.ds(0, _LO)]],
                    lobufs[s], out_hbm.at[pl.ds(r0, _LO)],
                    glsems[s], olsems[s])
        return (table_hbm.at[idx_v.at[j, pl.ds(_LO, _HI)]],
                hibufs[s], out_hbm.at[pl.ds(r0 + _LO, _HI)],
                ghsems[s], ohsems[s])

    def g_start(h, s, par):
        src, buf, _, gsem, _ = parts(h, s, par)
        pltpu.async_copy(src, buf, gsem)

    def g_wait(h, s, par):
        src, buf, _, gsem, _ = parts(h, s, par)
        pltpu.make_async_copy(src, buf, gsem).wait()

    def o_start(h, s, par):
        _, buf, dst, _, osem = parts(h, s, par)
        pltpu.async_copy(buf, dst, osem)

    def o_wait(h, s, par):
        _, buf, dst, _, osem = parts(h, s, par)
        pltpu.make_async_copy(buf, dst, osem).wait()

    def add_pe(s, par):
        buf = (lobufs if par == 0 else hibufs)[s]
        r0, nr = (0, _LO) if par == 0 else (_LO, _HI)

        @pl.loop(0, nr)
        def _row(r):
            for c in range(_D // 16):
                sl = pl.ds(c * 16, 16)
                plsc.addupdate(buf.at[r, sl], pe_v[r0 + r, sl])

    def body(h, s, par, *, wait_out=True, start_next=True):
        if wait_out:
            o_wait(h - 4, (s + 1) % 3, par)
        if start_next:
            g_start(h + 2, (s + 1) % 3, par)
        g_wait(h, s, par)
        add_pe(s, par)
        o_start(h, s, par)

    def sp(hh):
        return (hh // 2) % 3, hh % 2

    g_start(0, 0, 0)
    g_start(1, 0, 1)
    pe_copy.wait()
    for hh in range(4):
        body(hh, *sp(hh), wait_out=False)

    @pl.loop(4, _NH - 6, step=6)
    def _six(h):
        for off in range(6):
            body(h + off, *sp(4 + off))

    for hh in range(_NH - 6, _NH):
        body(hh, *sp(hh), start_next=(hh + 2 < _NH))
    for hh in range(_NH - 4, _NH):
        o_wait(hh, *sp(hh))


def kernel(x, table):
    pe = jnp.asarray(_pos_encoding_np())
    mesh = plsc.VectorSubcoreMesh(core_axis_name="c", subcore_axis_name="s")
    run = pl.kernel(
        _embed_body,
        out_type=jax.ShapeDtypeStruct((_B * _L, _D), jnp.float32),
        mesh=mesh,
        scratch_types=[
            pltpu.VMEM((_SENT_W, _L), jnp.int32),
            pltpu.VMEM((_L, _D), jnp.float32),
            pltpu.VMEM((_LO, _D), jnp.float32),
            pltpu.VMEM((_LO, _D), jnp.float32),
            pltpu.VMEM((_LO, _D), jnp.float32),
            pltpu.VMEM((_HI, _D), jnp.float32),
            pltpu.VMEM((_HI, _D), jnp.float32),
            pltpu.VMEM((_HI, _D), jnp.float32),
        ] + [pltpu.SemaphoreType.DMA] * 13,
    )
    return run(table, x.astype(jnp.int32), pe).reshape(_B, _L, _D)
